# baseline (speedup 1.0000x reference)
import jax
import jax.numpy as jnp
from jax.experimental import pallas as pl
from jax.experimental.pallas import tpu as pltpu
from jax.experimental.pallas import tpu_sc as plsc

_B, _S, _D = 1, 2048, 1024
_H, _Dh = 16, 64
_E, _K, _F = 8, 2, 1408
_EPS = 1e-06
_JITTER = 0.01
_TB = 256
_NTB = _S // _TB
_CK = 512
_NBLK = 24
_SLOTS = _NBLK * _TB
_W = 128
_QD = _D // 4


def _rms(x, g):
    v = jnp.mean(x * x, axis=-1, keepdims=True)
    return x * jax.lax.rsqrt(v + _EPS) * g


def _mm(a, b_t, out_dtype=jnp.float32):
    return jax.lax.dot_general(a, b_t, (((1,), (1,)), ((), ())),
                               preferred_element_type=out_dtype)


def _qkv_kernel(hs_ref, g1_ref, wq_ref, wk_ref, wv_ref, bq_ref, bk_ref,
                bv_ref, cos_ref, sin_ref, q_ref, k_ref, v_ref):
    x = hs_ref[...]
    h = _rms(x, g1_ref[...])
    c = cos_ref[...][:, None, :]
    s = sin_ref[...][:, None, :]
    sgn = jnp.where(
        jax.lax.broadcasted_iota(jnp.int32, (1, 1, _Dh), 2) < (_Dh // 2),
        -1.0, 1.0).astype(jnp.float32)

    def rope(y):
        y3 = y.reshape(_TB, _H, _Dh)
        rot = jnp.roll(y3, _Dh // 2, axis=-1) * sgn
        return (y3 * c + rot * s).transpose(1, 0, 2)

    q = _mm(h, wq_ref[...]) + bq_ref[...]
    k = _mm(h, wk_ref[...]) + bk_ref[...]
    v = _mm(h, wv_ref[...]) + bv_ref[...]
    q_ref[...] = rope(q)
    k_ref[...] = rope(k)
    v_ref[...] = v.reshape(_TB, _H, _Dh).transpose(1, 0, 2)


def _attn_kernel(q_ref, k_ref, v_ref, o_ref):
    qb = pl.program_id(1)
    q = q_ref[0]
    scale = 1.0 / (_Dh ** 0.5)
    nfull = qb // 2
    shift = qb * _TB - nfull * _CK

    def update(s, vc, carry):
        m_p, l_p, acc_p = carry
        m_n = jnp.maximum(m_p, jnp.max(s, axis=-1, keepdims=True))
        p = jnp.exp(s - m_n)
        corr = jnp.exp(m_p - m_n)
        l_n = l_p * corr + jnp.sum(p, axis=-1, keepdims=True)
        acc_n = acc_p * corr + jax.lax.dot_general(
            p, vc, (((1,), (0,)), ((), ())), preferred_element_type=jnp.float32)
        return m_n, l_n, acc_n

    def body(c, carry):
        m_p, l_p, acc_p, s_cur = carry
        k_n = k_ref[0, pl.ds((c + 1) * _CK, _CK), :]
        s_next = _mm(q, k_n) * scale
        v_c = v_ref[0, pl.ds(c * _CK, _CK), :]
        m_n, l_n, acc_n = update(s_cur, v_c, (m_p, l_p, acc_p))
        return m_n, l_n, acc_n, s_next

    s0 = _mm(q, k_ref[0, 0:_CK, :]) * scale
    init = (jnp.full((_TB, 1), -1e30, jnp.float32),
            jnp.zeros((_TB, 1), jnp.float32),
            jnp.zeros((_TB, _Dh), jnp.float32),
            s0)
    m_p, l_p, acc_p, s_diag = jax.lax.fori_loop(0, nfull, body, init)
    vd = v_ref[0, pl.ds(nfull * _CK, _CK), :]
    col = jax.lax.broadcasted_iota(jnp.int32, (_TB, _CK), 1)
    rowi = jax.lax.broadcasted_iota(jnp.int32, (_TB, _CK), 0)
    s = jnp.where(col <= rowi + shift, s_diag, jnp.float32(-1e9))
    m, l, acc = update(s, vd, (m_p, l_p, acc_p))
    o_ref[0] = acc / l


def _post_kernel(ctx_ref, res_ref, wo_ref, g2_ref, wr_ref, noise_ref,
                 h_ref, x_ref, r1_ref, r2_ref, e1_ref, e2_ref,
                 w1_ref, w2_ref, cnt_ref, aux_ref, carry_ref):
    i = pl.program_id(0)
    attn_out = jnp.zeros((_TB, _D), jnp.float32)
    for hh in range(_H):
        attn_out = attn_out + jax.lax.dot_general(
            ctx_ref[hh], wo_ref[hh], (((1,), (0,)), ((), ())),
            preferred_element_type=jnp.float32)
    h = attn_out + res_ref[...]
    h_ref[...] = h
    x = _rms(h, g2_ref[...])
    x_ref[...] = x
    logits = _mm(x, wr_ref[...]) + noise_ref[...]
    m = jnp.max(logits, axis=-1, keepdims=True)
    p = jnp.exp(logits - m)
    probs = p / jnp.sum(p, axis=-1, keepdims=True)
    lane = jax.lax.broadcasted_iota(jnp.int32, (_TB, _E), 1)
    i1 = jnp.argmax(probs, axis=-1)[:, None]
    oh1 = (lane == i1)
    v1 = jnp.max(probs, axis=-1, keepdims=True)
    probs2 = jnp.where(oh1, -1.0, probs)
    i2 = jnp.argmax(probs2, axis=-1)[:, None]
    oh2 = (lane == i2)
    v2 = jnp.max(probs2, axis=-1, keepdims=True)
    wsum = v1 + v2
    w1_ref[...] = v1 / wsum
    w2_ref[...] = v2 / wsum

    @pl.when(i == 0)
    def _():
        carry_ref[...] = jnp.zeros((1, _E), jnp.float32)

    ohb = (oh1 | oh2).astype(jnp.float32)
    ri = jax.lax.broadcasted_iota(jnp.int32, (_TB, _TB), 0)
    ci = jax.lax.broadcasted_iota(jnp.int32, (_TB, _TB), 1)
    tri = jnp.where(ri >= ci, 1.0, 0.0).astype(jnp.float32)
    incl = jax.lax.dot_general(tri, ohb, (((1,), (0,)), ((), ())),
                               preferred_element_type=jnp.float32)
    excl = incl - ohb + carry_ref[...]
    rank1 = jnp.sum(excl * oh1.astype(jnp.float32), axis=-1, keepdims=True)
    rank2 = jnp.sum(excl * oh2.astype(jnp.float32), axis=-1, keepdims=True)
    carry_ref[...] += jnp.sum(ohb, axis=0, keepdims=True)
    cnt_ref[...] = carry_ref[...]

    r1_ref[...] = rank1.reshape(1, _TB)
    r2_ref[...] = rank2.reshape(1, _TB)
    e1_ref[...] = i1.astype(jnp.float32).reshape(1, _TB)
    e2_ref[...] = i2.astype(jnp.float32).reshape(1, _TB)

    part = jnp.sum(probs, axis=0, keepdims=True)

    @pl.when(i == 0)
    def _():
        aux_ref[...] = part

    @pl.when(i > 0)
    def _():
        aux_ref[...] += part


def _slot_kernel(cnt_ref, r1_ref, r2_ref, e1_ref, e2_ref,
                 p1_ref, p2_ref, bexp_ref):
    off = []
    acc = jnp.float32(0.0)
    starts = []
    for e in range(_E):
        starts.append(acc)
        off.append(acc * _TB)
        acc = acc + jnp.ceil(cnt_ref[0, e] / _TB)
    r1 = r1_ref[...]
    r2 = r2_ref[...]
    e1 = e1_ref[...]
    e2 = e2_ref[...]
    p1 = r1
    p2 = r2
    for e in range(_E):
        p1 = p1 + jnp.where(e1 == e, off[e], 0.0)
        p2 = p2 + jnp.where(e2 == e, off[e], 0.0)
    p1_ref[...] = p1.astype(jnp.int32)
    p2_ref[...] = p2.astype(jnp.int32)
    bi = jax.lax.broadcasted_iota(jnp.int32, (1, _NBLK), 1).astype(jnp.float32)
    be = jnp.zeros((1, _NBLK), jnp.float32)
    for e in range(1, _E):
        be = be + jnp.where(bi >= starts[e], 1.0, 0.0)
    bexp_ref[...] = be.astype(jnp.int32)


def _sc_scatter(xq, pos1, pos2):
    mesh = plsc.VectorSubcoreMesh(core_axis_name="core",
                                  subcore_axis_name="subcore")
    otype = [jax.ShapeDtypeStruct((_SLOTS, _QD), jnp.float32)] * 4

    @pl.kernel(out_type=otype, mesh=mesh)
    def k(x0, x1, x2, x3, p1_hbm, p2_hbm, g0, g1, g2, g3):
        for x_hbm, xg_hbm in zip((x0, x1, x2, x3), (g0, g1, g2, g3)):
            def body(x_vmem, p1_vmem, p2_vmem, xg=xg_hbm):
                pltpu.sync_copy(x_vmem, xg.at[p1_vmem.at[0]])
                pltpu.sync_copy(x_vmem, xg.at[p2_vmem.at[0]])

            pltpu.emit_pipeline(
                body,
                grid=(_S // _W,),
                in_specs=[
                    pl.BlockSpec((_W, _QD), lambda i: (i, 0)),
                    pl.BlockSpec((1, _W), lambda i: (0, i)),
                    pl.BlockSpec((1, _W), lambda i: (0, i)),
                ],
                out_specs=[],
                core_axis_name='subcore',
                dimension_semantics=(pltpu.PARALLEL,),
            )(x_hbm, p1_hbm, p2_hbm)

    return k(*xq, pos1, pos2)


def _sc_gather(ygq, pos1, pos2):
    mesh = plsc.VectorSubcoreMesh(core_axis_name="core",
                                  subcore_axis_name="subcore")
    otype = [jax.ShapeDtypeStruct((_S, _QD), jnp.float32)] * 8

    @pl.kernel(out_type=otype, mesh=mesh)
    def k(y0, y1, y2, y3, p1_hbm, p2_hbm, *outs):
        for q, yg_hbm in enumerate((y0, y1, y2, y3)):
            for j, p_hbm in enumerate((p1_hbm, p2_hbm)):
                def body(p_vmem, g_vmem, yg=yg_hbm):
                    pltpu.sync_copy(yg.at[p_vmem.at[0]], g_vmem)

                pltpu.emit_pipeline(
                    body,
                    grid=(_S // _W,),
                    in_specs=[pl.BlockSpec((1, _W), lambda i: (0, i))],
                    out_specs=[pl.BlockSpec((_W, _QD), lambda i: (i, 0))],
                    core_axis_name='subcore',
                    dimension_semantics=(pltpu.PARALLEL,),
                )(p_hbm, outs[2 * q + j])

    return k(*ygq, pos1, pos2)


def _ffn_kernel(bexp_ref, x0_ref, x1_ref, x2_ref, x3_ref,
                wg_ref, wu_ref, wd_ref,
                y0_ref, y1_ref, y2_ref, y3_ref):
    x = jnp.concatenate(
        [x0_ref[...], x1_ref[...], x2_ref[...], x3_ref[...]],
        axis=1).astype(jnp.bfloat16)
    g = _mm(x, wg_ref[0])
    u = _mm(x, wu_ref[0])
    a = (g * jax.lax.logistic(g) * u).astype(jnp.bfloat16)
    y = _mm(a, wd_ref[0])
    y0_ref[...] = y[:, 0 * _QD:1 * _QD]
    y1_ref[...] = y[:, 1 * _QD:2 * _QD]
    y2_ref[...] = y[:, 2 * _QD:3 * _QD]
    y3_ref[...] = y[:, 3 * _QD:4 * _QD]


def _combine_kernel(h_ref, w1_ref, w2_ref, *g_refs):
    gq, out_ref = g_refs[:8], g_refs[8]
    w1 = w1_ref[...]
    w2 = w2_ref[...]
    parts = []
    for q in range(4):
        g1 = gq[2 * q][...]
        g2 = gq[2 * q + 1][...]
        parts.append(w1 * g1 + w2 * g2)
    out_ref[...] = h_ref[...] + jnp.concatenate(parts, axis=1)


def kernel(hidden_states, cos, sin, g1, g2, Wq, bq, Wk, bk, Wv, bv, Wo,
           Wr, Wg, Wu, Wd):
    hs = hidden_states.reshape(_S, _D)
    cos2 = cos.reshape(_S, _Dh)
    sin2 = sin.reshape(_S, _Dh)
    g1r = g1.reshape(1, _D)
    g2r = g2.reshape(1, _D)
    bqr = bq.reshape(1, _D)
    bkr = bk.reshape(1, _D)
    bvr = bv.reshape(1, _D)
    noise = (jax.random.normal(jax.random.key(42), (_S, _E), jnp.float32)
             * _JITTER)

    f32 = jnp.float32
    qkv_shapes = [jax.ShapeDtypeStruct((_H, _S, _Dh), f32)] * 3
    q, k, v = pl.pallas_call(
        _qkv_kernel,
        grid=(_NTB,),
        in_specs=[
            pl.BlockSpec((_TB, _D), lambda i: (i, 0)),
            pl.BlockSpec((1, _D), lambda i: (0, 0)),
            pl.BlockSpec((_D, _D), lambda i: (0, 0)),
            pl.BlockSpec((_D, _D), lambda i: (0, 0)),
            pl.BlockSpec((_D, _D), lambda i: (0, 0)),
            pl.BlockSpec((1, _D), lambda i: (0, 0)),
            pl.BlockSpec((1, _D), lambda i: (0, 0)),
            pl.BlockSpec((1, _D), lambda i: (0, 0)),
            pl.BlockSpec((_TB, _Dh), lambda i: (i, 0)),
            pl.BlockSpec((_TB, _Dh), lambda i: (i, 0)),
        ],
        out_specs=[pl.BlockSpec((_H, _TB, _Dh), lambda i: (0, i, 0))] * 3,
        out_shape=qkv_shapes,
    )(hs, g1r, Wq, Wk, Wv, bqr, bkr, bvr, cos2, sin2)

    ctx = pl.pallas_call(
        _attn_kernel,
        grid=(_H, _NTB),
        in_specs=[
            pl.BlockSpec((1, _TB, _Dh), lambda h, i: (h, i, 0)),
            pl.BlockSpec((1, _S, _Dh), lambda h, i: (h, 0, 0)),
            pl.BlockSpec((1, _S, _Dh), lambda h, i: (h, 0, 0)),
        ],
        out_specs=pl.BlockSpec((1, _TB, _Dh), lambda h, i: (h, i, 0)),
        out_shape=jax.ShapeDtypeStruct((_H, _S, _Dh), f32),
    )(q, k, v)

    wo3 = Wo.reshape(_D, _H, _Dh).transpose(1, 2, 0)

    (h_res, x_f, r1, r2, e1, e2, w1c, w2c, cnt, aux_part) = pl.pallas_call(
        _post_kernel,
        grid=(_NTB,),
        in_specs=[
            pl.BlockSpec((_H, _TB, _Dh), lambda i: (0, i, 0)),
            pl.BlockSpec((_TB, _D), lambda i: (i, 0)),
            pl.BlockSpec((_H, _Dh, _D), lambda i: (0, 0, 0)),
            pl.BlockSpec((1, _D), lambda i: (0, 0)),
            pl.BlockSpec((_E, _D), lambda i: (0, 0)),
            pl.BlockSpec((_TB, _E), lambda i: (i, 0)),
        ],
        out_specs=[
            pl.BlockSpec((_TB, _D), lambda i: (i, 0)),
            pl.BlockSpec((_TB, _D), lambda i: (i, 0)),
            pl.BlockSpec((1, _TB), lambda i: (0, i)),
            pl.BlockSpec((1, _TB), lambda i: (0, i)),
            pl.BlockSpec((1, _TB), lambda i: (0, i)),
            pl.BlockSpec((1, _TB), lambda i: (0, i)),
            pl.BlockSpec((_TB, 1), lambda i: (i, 0)),
            pl.BlockSpec((_TB, 1), lambda i: (i, 0)),
            pl.BlockSpec((1, _E), lambda i: (0, 0)),
            pl.BlockSpec((1, _E), lambda i: (0, 0)),
        ],
        out_shape=[
            jax.ShapeDtypeStruct((_S, _D), f32),
            jax.ShapeDtypeStruct((_S, _D), f32),
            jax.ShapeDtypeStruct((1, _S), f32),
            jax.ShapeDtypeStruct((1, _S), f32),
            jax.ShapeDtypeStruct((1, _S), f32),
            jax.ShapeDtypeStruct((1, _S), f32),
            jax.ShapeDtypeStruct((_S, 1), f32),
            jax.ShapeDtypeStruct((_S, 1), f32),
            jax.ShapeDtypeStruct((1, _E), f32),
            jax.ShapeDtypeStruct((1, _E), f32),
        ],
        scratch_shapes=[pltpu.VMEM((1, _E), f32)],
    )(ctx, hs, wo3, g2r, Wr, noise)

    pos1, pos2, bexp = pl.pallas_call(
        _slot_kernel,
        grid=(1,),
        in_specs=[
            pl.BlockSpec((1, _E), lambda i: (0, 0)),
            pl.BlockSpec((1, _S), lambda i: (0, 0)),
            pl.BlockSpec((1, _S), lambda i: (0, 0)),
            pl.BlockSpec((1, _S), lambda i: (0, 0)),
            pl.BlockSpec((1, _S), lambda i: (0, 0)),
        ],
        out_specs=[
            pl.BlockSpec((1, _S), lambda i: (0, 0)),
            pl.BlockSpec((1, _S), lambda i: (0, 0)),
            pl.BlockSpec((1, _NBLK), lambda i: (0, 0)),
        ],
        out_shape=[
            jax.ShapeDtypeStruct((1, _S), jnp.int32),
            jax.ShapeDtypeStruct((1, _S), jnp.int32),
            jax.ShapeDtypeStruct((1, _NBLK), jnp.int32),
        ],
    )(cnt, r1, r2, e1, e2)

    xq = [x_f[:, q * _QD:(q + 1) * _QD] for q in range(4)]
    xgq = _sc_scatter(xq, pos1, pos2)

    wg_b = Wg.astype(jnp.bfloat16)
    wu_b = Wu.astype(jnp.bfloat16)
    wd_b = Wd.astype(jnp.bfloat16)

    qspec = pl.BlockSpec((_TB, _QD), lambda b, s: (b, 0))
    ygq = pl.pallas_call(
        _ffn_kernel,
        grid_spec=pltpu.PrefetchScalarGridSpec(
            num_scalar_prefetch=1,
            grid=(_NBLK,),
            in_specs=[
                qspec, qspec, qspec, qspec,
                pl.BlockSpec((1, _F, _D), lambda b, s: (s[b], 0, 0)),
                pl.BlockSpec((1, _F, _D), lambda b, s: (s[b], 0, 0)),
                pl.BlockSpec((1, _D, _F), lambda b, s: (s[b], 0, 0)),
            ],
            out_specs=[qspec, qspec, qspec, qspec],
        ),
        out_shape=[jax.ShapeDtypeStruct((_SLOTS, _QD), jnp.float32)] * 4,
    )(bexp.reshape(_NBLK), *xgq, wg_b, wu_b, wd_b)

    gq = _sc_gather(ygq, pos1, pos2)

    gspec = pl.BlockSpec((_TB, _QD), lambda i: (i, 0))
    out2d = pl.pallas_call(
        _combine_kernel,
        grid=(_NTB,),
        in_specs=[
            pl.BlockSpec((_TB, _D), lambda i: (i, 0)),
            pl.BlockSpec((_TB, 1), lambda i: (i, 0)),
            pl.BlockSpec((_TB, 1), lambda i: (i, 0)),
        ] + [gspec] * 8,
        out_specs=pl.BlockSpec((_TB, _D), lambda i: (i, 0)),
        out_shape=jax.ShapeDtypeStruct((_S, _D), f32),
    )(h_res, w1c, w2c, *gq)

    aux_loss = jnp.mean(_E * (aux_part[0] / _S) ** 2)
    return out2d.reshape(_B, _S, _D), aux_loss


# speedup vs baseline: 1.0109x; 1.0109x over previous
import jax
import jax.numpy as jnp
from jax.experimental import pallas as pl
from jax.experimental.pallas import tpu as pltpu
from jax.experimental.pallas import tpu_sc as plsc

_B, _S, _D = 1, 2048, 1024
_H, _Dh = 16, 64
_E, _K, _F = 8, 2, 1408
_EPS = 1e-06
_JITTER = 0.01
_TB = 256
_NTB = _S // _TB
_CK = 512
_NBLK = 24
_SLOTS = _NBLK * _TB
_W = 128
_QD = _D // 4


def _rms(x, g):
    v = jnp.mean(x * x, axis=-1, keepdims=True)
    return x * jax.lax.rsqrt(v + _EPS) * g


def _mm(a, b_t, out_dtype=jnp.float32):
    return jax.lax.dot_general(a, b_t, (((1,), (1,)), ((), ())),
                               preferred_element_type=out_dtype)


def _qkv_kernel(hs_ref, g1_ref, wq_ref, wk_ref, wv_ref, bq_ref, bk_ref,
                bv_ref, cos_ref, sin_ref, q_ref, k_ref, v_ref):
    x = hs_ref[...]
    h = _rms(x, g1_ref[...])
    c = cos_ref[...][:, None, :]
    s = sin_ref[...][:, None, :]
    sgn = jnp.where(
        jax.lax.broadcasted_iota(jnp.int32, (1, 1, _Dh), 2) < (_Dh // 2),
        -1.0, 1.0).astype(jnp.float32)

    def rope(y):
        y3 = y.reshape(_TB, _H, _Dh)
        rot = jnp.roll(y3, _Dh // 2, axis=-1) * sgn
        return (y3 * c + rot * s).transpose(1, 0, 2)

    q = _mm(h, wq_ref[...]) + bq_ref[...]
    k = _mm(h, wk_ref[...]) + bk_ref[...]
    v = _mm(h, wv_ref[...]) + bv_ref[...]
    q_ref[...] = rope(q)
    k_ref[...] = rope(k)
    v_ref[...] = v.reshape(_TB, _H, _Dh).transpose(1, 0, 2)


def _attn_kernel(q_ref, k_ref, v_ref, o_ref):
    qb = pl.program_id(1)
    q = q_ref[0]
    scale = 1.0 / (_Dh ** 0.5)
    nfull = qb // 2
    shift = qb * _TB - nfull * _CK

    def update(s, vc, carry):
        m_p, l_p, acc_p = carry
        m_n = jnp.maximum(m_p, jnp.max(s, axis=-1, keepdims=True))
        p = jnp.exp(s - m_n)
        corr = jnp.exp(m_p - m_n)
        l_n = l_p * corr + jnp.sum(p, axis=-1, keepdims=True)
        acc_n = acc_p * corr + jax.lax.dot_general(
            p, vc, (((1,), (0,)), ((), ())), preferred_element_type=jnp.float32)
        return m_n, l_n, acc_n

    def body(c, carry):
        m_p, l_p, acc_p, s_cur = carry
        k_n = k_ref[0, pl.ds((c + 1) * _CK, _CK), :]
        s_next = _mm(q, k_n) * scale
        v_c = v_ref[0, pl.ds(c * _CK, _CK), :]
        m_n, l_n, acc_n = update(s_cur, v_c, (m_p, l_p, acc_p))
        return m_n, l_n, acc_n, s_next

    s0 = _mm(q, k_ref[0, 0:_CK, :]) * scale
    init = (jnp.full((_TB, 1), -1e30, jnp.float32),
            jnp.zeros((_TB, 1), jnp.float32),
            jnp.zeros((_TB, _Dh), jnp.float32),
            s0)
    m_p, l_p, acc_p, s_diag = jax.lax.fori_loop(0, nfull, body, init)
    vd = v_ref[0, pl.ds(nfull * _CK, _CK), :]
    col = jax.lax.broadcasted_iota(jnp.int32, (_TB, _CK), 1)
    rowi = jax.lax.broadcasted_iota(jnp.int32, (_TB, _CK), 0)
    s = jnp.where(col <= rowi + shift, s_diag, jnp.float32(-1e9))
    m, l, acc = update(s, vd, (m_p, l_p, acc_p))
    o_ref[0] = acc / l


def _post_kernel(ctx_ref, res_ref, wo_ref, g2_ref, wr_ref, noise_ref,
                 h_ref, x_ref, r1_ref, r2_ref, e1_ref, e2_ref,
                 w1_ref, w2_ref, cnt_ref, aux_ref, carry_ref):
    i = pl.program_id(0)
    attn_out = _mm(ctx_ref[...], wo_ref[...])
    h = attn_out + res_ref[...]
    h_ref[...] = h
    x = _rms(h, g2_ref[...])
    x_ref[...] = x
    logits = _mm(x, wr_ref[...]) + noise_ref[...]
    m = jnp.max(logits, axis=-1, keepdims=True)
    p = jnp.exp(logits - m)
    probs = p / jnp.sum(p, axis=-1, keepdims=True)
    lane = jax.lax.broadcasted_iota(jnp.int32, (_TB, _E), 1)
    i1 = jnp.argmax(probs, axis=-1)[:, None]
    oh1 = (lane == i1)
    v1 = jnp.max(probs, axis=-1, keepdims=True)
    probs2 = jnp.where(oh1, -1.0, probs)
    i2 = jnp.argmax(probs2, axis=-1)[:, None]
    oh2 = (lane == i2)
    v2 = jnp.max(probs2, axis=-1, keepdims=True)
    wsum = v1 + v2
    w1_ref[...] = v1 / wsum
    w2_ref[...] = v2 / wsum

    @pl.when(i == 0)
    def _():
        carry_ref[...] = jnp.zeros((1, _E), jnp.float32)

    ohb = (oh1 | oh2).astype(jnp.float32)
    ri = jax.lax.broadcasted_iota(jnp.int32, (_TB, _TB), 0)
    ci = jax.lax.broadcasted_iota(jnp.int32, (_TB, _TB), 1)
    tri = jnp.where(ri >= ci, 1.0, 0.0).astype(jnp.float32)
    incl = jax.lax.dot_general(tri, ohb, (((1,), (0,)), ((), ())),
                               preferred_element_type=jnp.float32)
    excl = incl - ohb + carry_ref[...]
    rank1 = jnp.sum(excl * oh1.astype(jnp.float32), axis=-1, keepdims=True)
    rank2 = jnp.sum(excl * oh2.astype(jnp.float32), axis=-1, keepdims=True)
    carry_ref[...] += jnp.sum(ohb, axis=0, keepdims=True)
    cnt_ref[...] = carry_ref[...]

    r1_ref[...] = rank1.reshape(1, _TB)
    r2_ref[...] = rank2.reshape(1, _TB)
    e1_ref[...] = i1.astype(jnp.float32).reshape(1, _TB)
    e2_ref[...] = i2.astype(jnp.float32).reshape(1, _TB)

    part = jnp.sum(probs, axis=0, keepdims=True)

    @pl.when(i == 0)
    def _():
        aux_ref[...] = part

    @pl.when(i > 0)
    def _():
        aux_ref[...] += part


def _slot_kernel(cnt_ref, r1_ref, r2_ref, e1_ref, e2_ref,
                 p1_ref, p2_ref, bexp_ref):
    off = []
    acc = jnp.float32(0.0)
    starts = []
    for e in range(_E):
        starts.append(acc)
        off.append(acc * _TB)
        acc = acc + jnp.ceil(cnt_ref[0, e] / _TB)
    r1 = r1_ref[...]
    r2 = r2_ref[...]
    e1 = e1_ref[...]
    e2 = e2_ref[...]
    p1 = r1
    p2 = r2
    for e in range(_E):
        p1 = p1 + jnp.where(e1 == e, off[e], 0.0)
        p2 = p2 + jnp.where(e2 == e, off[e], 0.0)
    p1_ref[...] = p1.astype(jnp.int32)
    p2_ref[...] = p2.astype(jnp.int32)
    bi = jax.lax.broadcasted_iota(jnp.int32, (1, _NBLK), 1).astype(jnp.float32)
    be = jnp.zeros((1, _NBLK), jnp.float32)
    for e in range(1, _E):
        be = be + jnp.where(bi >= starts[e], 1.0, 0.0)
    bexp_ref[...] = be.astype(jnp.int32)


def _sc_scatter(xq, pos1, pos2):
    mesh = plsc.VectorSubcoreMesh(core_axis_name="core",
                                  subcore_axis_name="subcore")
    otype = [jax.ShapeDtypeStruct((_SLOTS, _QD), jnp.float32)] * 4

    @pl.kernel(out_type=otype, mesh=mesh)
    def k(x0, x1, x2, x3, p1_hbm, p2_hbm, g0, g1, g2, g3):
        for x_hbm, xg_hbm in zip((x0, x1, x2, x3), (g0, g1, g2, g3)):
            def body(x_vmem, p1_vmem, p2_vmem, xg=xg_hbm):
                pltpu.sync_copy(x_vmem, xg.at[p1_vmem.at[0]])
                pltpu.sync_copy(x_vmem, xg.at[p2_vmem.at[0]])

            pltpu.emit_pipeline(
                body,
                grid=(_S // _W,),
                in_specs=[
                    pl.BlockSpec((_W, _QD), lambda i: (i, 0)),
                    pl.BlockSpec((1, _W), lambda i: (0, i)),
                    pl.BlockSpec((1, _W), lambda i: (0, i)),
                ],
                out_specs=[],
                core_axis_name='subcore',
                dimension_semantics=(pltpu.PARALLEL,),
            )(x_hbm, p1_hbm, p2_hbm)

    return k(*xq, pos1, pos2)


def _sc_gather(ygq, pos1, pos2):
    mesh = plsc.VectorSubcoreMesh(core_axis_name="core",
                                  subcore_axis_name="subcore")
    otype = [jax.ShapeDtypeStruct((_S, _QD), jnp.float32)] * 8

    @pl.kernel(out_type=otype, mesh=mesh)
    def k(y0, y1, y2, y3, p1_hbm, p2_hbm, *outs):
        for q, yg_hbm in enumerate((y0, y1, y2, y3)):
            for j, p_hbm in enumerate((p1_hbm, p2_hbm)):
                def body(p_vmem, g_vmem, yg=yg_hbm):
                    pltpu.sync_copy(yg.at[p_vmem.at[0]], g_vmem)

                pltpu.emit_pipeline(
                    body,
                    grid=(_S // _W,),
                    in_specs=[pl.BlockSpec((1, _W), lambda i: (0, i))],
                    out_specs=[pl.BlockSpec((_W, _QD), lambda i: (i, 0))],
                    core_axis_name='subcore',
                    dimension_semantics=(pltpu.PARALLEL,),
                )(p_hbm, outs[2 * q + j])

    return k(*ygq, pos1, pos2)


def _ffn_kernel(bexp_ref, x0_ref, x1_ref, x2_ref, x3_ref,
                wg_ref, wu_ref, wd_ref,
                y0_ref, y1_ref, y2_ref, y3_ref):
    x = jnp.concatenate(
        [x0_ref[...], x1_ref[...], x2_ref[...], x3_ref[...]],
        axis=1).astype(jnp.bfloat16)
    g = _mm(x, wg_ref[0])
    u = _mm(x, wu_ref[0])
    a = (g * jax.lax.logistic(g) * u).astype(jnp.bfloat16)
    y = _mm(a, wd_ref[0])
    y0_ref[...] = y[:, 0 * _QD:1 * _QD]
    y1_ref[...] = y[:, 1 * _QD:2 * _QD]
    y2_ref[...] = y[:, 2 * _QD:3 * _QD]
    y3_ref[...] = y[:, 3 * _QD:4 * _QD]


def _combine_kernel(h_ref, w1_ref, w2_ref, *g_refs):
    gq, out_ref = g_refs[:8], g_refs[8]
    w1 = w1_ref[...]
    w2 = w2_ref[...]
    parts = []
    for q in range(4):
        g1 = gq[2 * q][...]
        g2 = gq[2 * q + 1][...]
        parts.append(w1 * g1 + w2 * g2)
    out_ref[...] = h_ref[...] + jnp.concatenate(parts, axis=1)


def kernel(hidden_states, cos, sin, g1, g2, Wq, bq, Wk, bk, Wv, bv, Wo,
           Wr, Wg, Wu, Wd):
    hs = hidden_states.reshape(_S, _D)
    cos2 = cos.reshape(_S, _Dh)
    sin2 = sin.reshape(_S, _Dh)
    g1r = g1.reshape(1, _D)
    g2r = g2.reshape(1, _D)
    bqr = bq.reshape(1, _D)
    bkr = bk.reshape(1, _D)
    bvr = bv.reshape(1, _D)
    noise = (jax.random.normal(jax.random.key(42), (_S, _E), jnp.float32)
             * _JITTER)

    f32 = jnp.float32
    qkv_shapes = [jax.ShapeDtypeStruct((_H, _S, _Dh), f32)] * 3
    q, k, v = pl.pallas_call(
        _qkv_kernel,
        grid=(_NTB,),
        in_specs=[
            pl.BlockSpec((_TB, _D), lambda i: (i, 0)),
            pl.BlockSpec((1, _D), lambda i: (0, 0)),
            pl.BlockSpec((_D, _D), lambda i: (0, 0)),
            pl.BlockSpec((_D, _D), lambda i: (0, 0)),
            pl.BlockSpec((_D, _D), lambda i: (0, 0)),
            pl.BlockSpec((1, _D), lambda i: (0, 0)),
            pl.BlockSpec((1, _D), lambda i: (0, 0)),
            pl.BlockSpec((1, _D), lambda i: (0, 0)),
            pl.BlockSpec((_TB, _Dh), lambda i: (i, 0)),
            pl.BlockSpec((_TB, _Dh), lambda i: (i, 0)),
        ],
        out_specs=[pl.BlockSpec((_H, _TB, _Dh), lambda i: (0, i, 0))] * 3,
        out_shape=qkv_shapes,
    )(hs, g1r, Wq, Wk, Wv, bqr, bkr, bvr, cos2, sin2)

    ctx = pl.pallas_call(
        _attn_kernel,
        grid=(_H, _NTB),
        in_specs=[
            pl.BlockSpec((1, _TB, _Dh), lambda h, i: (h, i, 0)),
            pl.BlockSpec((1, _S, _Dh), lambda h, i: (h, 0, 0)),
            pl.BlockSpec((1, _S, _Dh), lambda h, i: (h, 0, 0)),
        ],
        out_specs=pl.BlockSpec((1, _TB, _Dh), lambda h, i: (h, i, 0)),
        out_shape=jax.ShapeDtypeStruct((_H, _S, _Dh), f32),
    )(q, k, v)

    ctx2d = ctx.transpose(1, 0, 2).reshape(_S, _D)

    (h_res, x_f, r1, r2, e1, e2, w1c, w2c, cnt, aux_part) = pl.pallas_call(
        _post_kernel,
        grid=(_NTB,),
        in_specs=[
            pl.BlockSpec((_TB, _D), lambda i: (i, 0)),
            pl.BlockSpec((_TB, _D), lambda i: (i, 0)),
            pl.BlockSpec((_D, _D), lambda i: (0, 0)),
            pl.BlockSpec((1, _D), lambda i: (0, 0)),
            pl.BlockSpec((_E, _D), lambda i: (0, 0)),
            pl.BlockSpec((_TB, _E), lambda i: (i, 0)),
        ],
        out_specs=[
            pl.BlockSpec((_TB, _D), lambda i: (i, 0)),
            pl.BlockSpec((_TB, _D), lambda i: (i, 0)),
            pl.BlockSpec((1, _TB), lambda i: (0, i)),
            pl.BlockSpec((1, _TB), lambda i: (0, i)),
            pl.BlockSpec((1, _TB), lambda i: (0, i)),
            pl.BlockSpec((1, _TB), lambda i: (0, i)),
            pl.BlockSpec((_TB, 1), lambda i: (i, 0)),
            pl.BlockSpec((_TB, 1), lambda i: (i, 0)),
            pl.BlockSpec((1, _E), lambda i: (0, 0)),
            pl.BlockSpec((1, _E), lambda i: (0, 0)),
        ],
        out_shape=[
            jax.ShapeDtypeStruct((_S, _D), f32),
            jax.ShapeDtypeStruct((_S, _D), f32),
            jax.ShapeDtypeStruct((1, _S), f32),
            jax.ShapeDtypeStruct((1, _S), f32),
            jax.ShapeDtypeStruct((1, _S), f32),
            jax.ShapeDtypeStruct((1, _S), f32),
            jax.ShapeDtypeStruct((_S, 1), f32),
            jax.ShapeDtypeStruct((_S, 1), f32),
            jax.ShapeDtypeStruct((1, _E), f32),
            jax.ShapeDtypeStruct((1, _E), f32),
        ],
        scratch_shapes=[pltpu.VMEM((1, _E), f32)],
    )(ctx2d, hs, Wo, g2r, Wr, noise)

    pos1, pos2, bexp = pl.pallas_call(
        _slot_kernel,
        grid=(1,),
        in_specs=[
            pl.BlockSpec((1, _E), lambda i: (0, 0)),
            pl.BlockSpec((1, _S), lambda i: (0, 0)),
            pl.BlockSpec((1, _S), lambda i: (0, 0)),
            pl.BlockSpec((1, _S), lambda i: (0, 0)),
            pl.BlockSpec((1, _S), lambda i: (0, 0)),
        ],
        out_specs=[
            pl.BlockSpec((1, _S), lambda i: (0, 0)),
            pl.BlockSpec((1, _S), lambda i: (0, 0)),
            pl.BlockSpec((1, _NBLK), lambda i: (0, 0)),
        ],
        out_shape=[
            jax.ShapeDtypeStruct((1, _S), jnp.int32),
            jax.ShapeDtypeStruct((1, _S), jnp.int32),
            jax.ShapeDtypeStruct((1, _NBLK), jnp.int32),
        ],
    )(cnt, r1, r2, e1, e2)

    xq = [x_f[:, q * _QD:(q + 1) * _QD] for q in range(4)]
    xgq = _sc_scatter(xq, pos1, pos2)

    wg_b = Wg.astype(jnp.bfloat16)
    wu_b = Wu.astype(jnp.bfloat16)
    wd_b = Wd.astype(jnp.bfloat16)

    qspec = pl.BlockSpec((_TB, _QD), lambda b, s: (b, 0))
    ygq = pl.pallas_call(
        _ffn_kernel,
        grid_spec=pltpu.PrefetchScalarGridSpec(
            num_scalar_prefetch=1,
            grid=(_NBLK,),
            in_specs=[
                qspec, qspec, qspec, qspec,
                pl.BlockSpec((1, _F, _D), lambda b, s: (s[b], 0, 0)),
                pl.BlockSpec((1, _F, _D), lambda b, s: (s[b], 0, 0)),
                pl.BlockSpec((1, _D, _F), lambda b, s: (s[b], 0, 0)),
            ],
            out_specs=[qspec, qspec, qspec, qspec],
        ),
        out_shape=[jax.ShapeDtypeStruct((_SLOTS, _QD), jnp.float32)] * 4,
    )(bexp.reshape(_NBLK), *xgq, wg_b, wu_b, wd_b)

    gq = _sc_gather(ygq, pos1, pos2)

    gspec = pl.BlockSpec((_TB, _QD), lambda i: (i, 0))
    out2d = pl.pallas_call(
        _combine_kernel,
        grid=(_NTB,),
        in_specs=[
            pl.BlockSpec((_TB, _D), lambda i: (i, 0)),
            pl.BlockSpec((_TB, 1), lambda i: (i, 0)),
            pl.BlockSpec((_TB, 1), lambda i: (i, 0)),
        ] + [gspec] * 8,
        out_specs=pl.BlockSpec((_TB, _D), lambda i: (i, 0)),
        out_shape=jax.ShapeDtypeStruct((_S, _D), f32),
    )(h_res, w1c, w2c, *gq)

    aux_loss = jnp.mean(_E * (aux_part[0] / _S) ** 2)
    return out2d.reshape(_B, _S, _D), aux_loss


# speedup vs baseline: 1.1551x; 1.1426x over previous
import jax
import jax.numpy as jnp
from jax.experimental import pallas as pl
from jax.experimental.pallas import tpu as pltpu
from jax.experimental.pallas import tpu_sc as plsc

_B, _S, _D = 1, 2048, 1024
_H, _Dh = 16, 64
_E, _K, _F = 8, 2, 1408
_EPS = 1e-06
_JITTER = 0.01
_TB = 256
_NTB = _S // _TB
_CK = 512
_NBLK = 24
_SLOTS = _NBLK * _TB
_W = 128
_QD = _D // 4


def _rms(x, g):
    v = jnp.mean(x * x, axis=-1, keepdims=True)
    return x * jax.lax.rsqrt(v + _EPS) * g


def _mm(a, b_t, out_dtype=jnp.float32):
    return jax.lax.dot_general(a, b_t, (((1,), (1,)), ((), ())),
                               preferred_element_type=out_dtype)


def _qkv_kernel(hs_ref, g1_ref, wq_ref, wk_ref, wv_ref, bq_ref, bk_ref,
                bv_ref, cos_ref, sin_ref, q_ref, k_ref, v_ref):
    x = hs_ref[...]
    h = _rms(x, g1_ref[...])
    c = cos_ref[...][:, None, :]
    s = sin_ref[...][:, None, :]
    sgn = jnp.where(
        jax.lax.broadcasted_iota(jnp.int32, (1, 1, _Dh), 2) < (_Dh // 2),
        -1.0, 1.0).astype(jnp.float32)

    def rope(y):
        y3 = y.reshape(_TB, _H, _Dh)
        rot = jnp.roll(y3, _Dh // 2, axis=-1) * sgn
        return (y3 * c + rot * s).transpose(1, 0, 2)

    q = _mm(h, wq_ref[...]) + bq_ref[...]
    k = _mm(h, wk_ref[...]) + bk_ref[...]
    v = _mm(h, wv_ref[...]) + bv_ref[...]
    q_ref[...] = rope(q)
    k_ref[...] = rope(k)
    v_ref[...] = v.reshape(_TB, _H, _Dh).transpose(1, 0, 2)


_TBQ = 512


def _attn_kernel(q_ref, k_ref, v_ref, o_ref):
    qb = pl.program_id(1)
    q = q_ref[0]
    scale = 1.0 / (_Dh ** 0.5)

    def update(s, vc, carry):
        m_p, l_p, acc_p = carry
        m_n = jnp.maximum(m_p, jnp.max(s, axis=-1, keepdims=True))
        p = jnp.exp(s - m_n)
        corr = jnp.exp(m_p - m_n)
        l_n = l_p * corr + jnp.sum(p, axis=-1, keepdims=True)
        acc_n = acc_p * corr + jax.lax.dot_general(
            p, vc, (((1,), (0,)), ((), ())), preferred_element_type=jnp.float32)
        return m_n, l_n, acc_n

    def body(c, carry):
        k = k_ref[0, pl.ds(c * _CK, _CK), :]
        v = v_ref[0, pl.ds(c * _CK, _CK), :]
        return update(_mm(q, k) * scale, v, carry)

    init = (jnp.full((_TBQ, 1), -1e30, jnp.float32),
            jnp.zeros((_TBQ, 1), jnp.float32),
            jnp.zeros((_TBQ, _Dh), jnp.float32))
    carry = jax.lax.fori_loop(0, qb, body, init)
    kd = k_ref[0, pl.ds(qb * _CK, _CK), :]
    vd = v_ref[0, pl.ds(qb * _CK, _CK), :]
    s = _mm(q, kd) * scale
    diag = (jax.lax.broadcasted_iota(jnp.int32, (_TBQ, _CK), 1)
            <= jax.lax.broadcasted_iota(jnp.int32, (_TBQ, _CK), 0))
    s = jnp.where(diag, s, jnp.float32(-1e9))
    m, l, acc = update(s, vd, carry)
    o_ref[0] = acc / l


def _post_kernel(ctx_ref, res_ref, wo_ref, g2_ref, wr_ref, noise_ref,
                 h_ref, x_ref, r1_ref, r2_ref, e1_ref, e2_ref,
                 w1_ref, w2_ref, cnt_ref, aux_ref, carry_ref):
    i = pl.program_id(0)
    attn_out = _mm(ctx_ref[...], wo_ref[...])
    h = attn_out + res_ref[...]
    h_ref[...] = h
    x = _rms(h, g2_ref[...])
    x_ref[...] = x
    logits = _mm(x, wr_ref[...]) + noise_ref[...]
    m = jnp.max(logits, axis=-1, keepdims=True)
    p = jnp.exp(logits - m)
    probs = p / jnp.sum(p, axis=-1, keepdims=True)
    lane = jax.lax.broadcasted_iota(jnp.int32, (_TB, _E), 1)
    i1 = jnp.argmax(probs, axis=-1)[:, None]
    oh1 = (lane == i1)
    v1 = jnp.max(probs, axis=-1, keepdims=True)
    probs2 = jnp.where(oh1, -1.0, probs)
    i2 = jnp.argmax(probs2, axis=-1)[:, None]
    oh2 = (lane == i2)
    v2 = jnp.max(probs2, axis=-1, keepdims=True)
    wsum = v1 + v2
    w1_ref[...] = v1 / wsum
    w2_ref[...] = v2 / wsum

    @pl.when(i == 0)
    def _():
        carry_ref[...] = jnp.zeros((1, _E), jnp.float32)

    ohb = (oh1 | oh2).astype(jnp.float32)
    ri = jax.lax.broadcasted_iota(jnp.int32, (_TB, _TB), 0)
    ci = jax.lax.broadcasted_iota(jnp.int32, (_TB, _TB), 1)
    tri = jnp.where(ri >= ci, 1.0, 0.0).astype(jnp.float32)
    incl = jax.lax.dot_general(tri, ohb, (((1,), (0,)), ((), ())),
                               preferred_element_type=jnp.float32)
    excl = incl - ohb + carry_ref[...]
    rank1 = jnp.sum(excl * oh1.astype(jnp.float32), axis=-1, keepdims=True)
    rank2 = jnp.sum(excl * oh2.astype(jnp.float32), axis=-1, keepdims=True)
    carry_ref[...] += jnp.sum(ohb, axis=0, keepdims=True)
    cnt_ref[...] = carry_ref[...]

    r1_ref[...] = rank1.reshape(1, _TB)
    r2_ref[...] = rank2.reshape(1, _TB)
    e1_ref[...] = i1.astype(jnp.float32).reshape(1, _TB)
    e2_ref[...] = i2.astype(jnp.float32).reshape(1, _TB)

    part = jnp.sum(probs, axis=0, keepdims=True)

    @pl.when(i == 0)
    def _():
        aux_ref[...] = part

    @pl.when(i > 0)
    def _():
        aux_ref[...] += part


def _slot_kernel(cnt_ref, r1_ref, r2_ref, e1_ref, e2_ref,
                 p1_ref, p2_ref, bexp_ref):
    off = []
    acc = jnp.float32(0.0)
    starts = []
    for e in range(_E):
        starts.append(acc)
        off.append(acc * _TB)
        acc = acc + jnp.ceil(cnt_ref[0, e] / _TB)
    r1 = r1_ref[...]
    r2 = r2_ref[...]
    e1 = e1_ref[...]
    e2 = e2_ref[...]
    p1 = r1
    p2 = r2
    for e in range(_E):
        p1 = p1 + jnp.where(e1 == e, off[e], 0.0)
        p2 = p2 + jnp.where(e2 == e, off[e], 0.0)
    p1_ref[...] = p1.astype(jnp.int32)
    p2_ref[...] = p2.astype(jnp.int32)
    bi = jax.lax.broadcasted_iota(jnp.int32, (1, _NBLK), 1).astype(jnp.float32)
    be = jnp.zeros((1, _NBLK), jnp.float32)
    for e in range(1, _E):
        be = be + jnp.where(bi >= starts[e], 1.0, 0.0)
    bexp_ref[...] = be.astype(jnp.int32)


def _sc_scatter(xq, pos1, pos2):
    mesh = plsc.VectorSubcoreMesh(core_axis_name="core",
                                  subcore_axis_name="subcore")
    otype = [jax.ShapeDtypeStruct((_SLOTS, _QD), jnp.float32)] * 4

    @pl.kernel(out_type=otype, mesh=mesh)
    def k(x0, x1, x2, x3, p1_hbm, p2_hbm, g0, g1, g2, g3):
        for x_hbm, xg_hbm in zip((x0, x1, x2, x3), (g0, g1, g2, g3)):
            def body(x_vmem, p1_vmem, p2_vmem, xg=xg_hbm):
                pltpu.sync_copy(x_vmem, xg.at[p1_vmem.at[0]])
                pltpu.sync_copy(x_vmem, xg.at[p2_vmem.at[0]])

            pltpu.emit_pipeline(
                body,
                grid=(_S // _W,),
                in_specs=[
                    pl.BlockSpec((_W, _QD), lambda i: (i, 0)),
                    pl.BlockSpec((1, _W), lambda i: (0, i)),
                    pl.BlockSpec((1, _W), lambda i: (0, i)),
                ],
                out_specs=[],
                core_axis_name='subcore',
                dimension_semantics=(pltpu.PARALLEL,),
            )(x_hbm, p1_hbm, p2_hbm)

    return k(*xq, pos1, pos2)


def _sc_gather(ygq, pos1, pos2):
    mesh = plsc.VectorSubcoreMesh(core_axis_name="core",
                                  subcore_axis_name="subcore")
    otype = [jax.ShapeDtypeStruct((_S, _QD), jnp.float32)] * 8

    @pl.kernel(out_type=otype, mesh=mesh)
    def k(y0, y1, y2, y3, p1_hbm, p2_hbm, *outs):
        for q, yg_hbm in enumerate((y0, y1, y2, y3)):
            for j, p_hbm in enumerate((p1_hbm, p2_hbm)):
                def body(p_vmem, g_vmem, yg=yg_hbm):
                    pltpu.sync_copy(yg.at[p_vmem.at[0]], g_vmem)

                pltpu.emit_pipeline(
                    body,
                    grid=(_S // _W,),
                    in_specs=[pl.BlockSpec((1, _W), lambda i: (0, i))],
                    out_specs=[pl.BlockSpec((_W, _QD), lambda i: (i, 0))],
                    core_axis_name='subcore',
                    dimension_semantics=(pltpu.PARALLEL,),
                )(p_hbm, outs[2 * q + j])

    return k(*ygq, pos1, pos2)


def _ffn_kernel(bexp_ref, x0_ref, x1_ref, x2_ref, x3_ref,
                wg_ref, wu_ref, wd_ref,
                y0_ref, y1_ref, y2_ref, y3_ref):
    x = jnp.concatenate(
        [x0_ref[...], x1_ref[...], x2_ref[...], x3_ref[...]],
        axis=1).astype(jnp.bfloat16)
    g = _mm(x, wg_ref[0])
    u = _mm(x, wu_ref[0])
    a = (g * jax.lax.logistic(g) * u).astype(jnp.bfloat16)
    y = _mm(a, wd_ref[0])
    y0_ref[...] = y[:, 0 * _QD:1 * _QD]
    y1_ref[...] = y[:, 1 * _QD:2 * _QD]
    y2_ref[...] = y[:, 2 * _QD:3 * _QD]
    y3_ref[...] = y[:, 3 * _QD:4 * _QD]


def _combine_kernel(h_ref, w1_ref, w2_ref, *g_refs):
    gq, out_ref = g_refs[:8], g_refs[8]
    w1 = w1_ref[...]
    w2 = w2_ref[...]
    parts = []
    for q in range(4):
        g1 = gq[2 * q][...]
        g2 = gq[2 * q + 1][...]
        parts.append(w1 * g1 + w2 * g2)
    out_ref[...] = h_ref[...] + jnp.concatenate(parts, axis=1)


def kernel(hidden_states, cos, sin, g1, g2, Wq, bq, Wk, bk, Wv, bv, Wo,
           Wr, Wg, Wu, Wd):
    hs = hidden_states.reshape(_S, _D)
    cos2 = cos.reshape(_S, _Dh)
    sin2 = sin.reshape(_S, _Dh)
    g1r = g1.reshape(1, _D)
    g2r = g2.reshape(1, _D)
    bqr = bq.reshape(1, _D)
    bkr = bk.reshape(1, _D)
    bvr = bv.reshape(1, _D)
    noise = (jax.random.normal(jax.random.key(42), (_S, _E), jnp.float32)
             * _JITTER)

    f32 = jnp.float32
    qkv_shapes = [jax.ShapeDtypeStruct((_H, _S, _Dh), f32)] * 3
    q, k, v = pl.pallas_call(
        _qkv_kernel,
        grid=(_NTB,),
        in_specs=[
            pl.BlockSpec((_TB, _D), lambda i: (i, 0)),
            pl.BlockSpec((1, _D), lambda i: (0, 0)),
            pl.BlockSpec((_D, _D), lambda i: (0, 0)),
            pl.BlockSpec((_D, _D), lambda i: (0, 0)),
            pl.BlockSpec((_D, _D), lambda i: (0, 0)),
            pl.BlockSpec((1, _D), lambda i: (0, 0)),
            pl.BlockSpec((1, _D), lambda i: (0, 0)),
            pl.BlockSpec((1, _D), lambda i: (0, 0)),
            pl.BlockSpec((_TB, _Dh), lambda i: (i, 0)),
            pl.BlockSpec((_TB, _Dh), lambda i: (i, 0)),
        ],
        out_specs=[pl.BlockSpec((_H, _TB, _Dh), lambda i: (0, i, 0))] * 3,
        out_shape=qkv_shapes,
    )(hs, g1r, Wq, Wk, Wv, bqr, bkr, bvr, cos2, sin2)

    ctx = pl.pallas_call(
        _attn_kernel,
        grid=(_H, _S // _TBQ),
        in_specs=[
            pl.BlockSpec((1, _TBQ, _Dh), lambda h, i: (h, i, 0)),
            pl.BlockSpec((1, _S, _Dh), lambda h, i: (h, 0, 0)),
            pl.BlockSpec((1, _S, _Dh), lambda h, i: (h, 0, 0)),
        ],
        out_specs=pl.BlockSpec((1, _TBQ, _Dh), lambda h, i: (h, i, 0)),
        out_shape=jax.ShapeDtypeStruct((_H, _S, _Dh), f32),
    )(q, k, v)

    ctx2d = ctx.transpose(1, 0, 2).reshape(_S, _D)

    (h_res, x_f, r1, r2, e1, e2, w1c, w2c, cnt, aux_part) = pl.pallas_call(
        _post_kernel,
        grid=(_NTB,),
        in_specs=[
            pl.BlockSpec((_TB, _D), lambda i: (i, 0)),
            pl.BlockSpec((_TB, _D), lambda i: (i, 0)),
            pl.BlockSpec((_D, _D), lambda i: (0, 0)),
            pl.BlockSpec((1, _D), lambda i: (0, 0)),
            pl.BlockSpec((_E, _D), lambda i: (0, 0)),
            pl.BlockSpec((_TB, _E), lambda i: (i, 0)),
        ],
        out_specs=[
            pl.BlockSpec((_TB, _D), lambda i: (i, 0)),
            pl.BlockSpec((_TB, _D), lambda i: (i, 0)),
            pl.BlockSpec((1, _TB), lambda i: (0, i)),
            pl.BlockSpec((1, _TB), lambda i: (0, i)),
            pl.BlockSpec((1, _TB), lambda i: (0, i)),
            pl.BlockSpec((1, _TB), lambda i: (0, i)),
            pl.BlockSpec((_TB, 1), lambda i: (i, 0)),
            pl.BlockSpec((_TB, 1), lambda i: (i, 0)),
            pl.BlockSpec((1, _E), lambda i: (0, 0)),
            pl.BlockSpec((1, _E), lambda i: (0, 0)),
        ],
        out_shape=[
            jax.ShapeDtypeStruct((_S, _D), f32),
            jax.ShapeDtypeStruct((_S, _D), f32),
            jax.ShapeDtypeStruct((1, _S), f32),
            jax.ShapeDtypeStruct((1, _S), f32),
            jax.ShapeDtypeStruct((1, _S), f32),
            jax.ShapeDtypeStruct((1, _S), f32),
            jax.ShapeDtypeStruct((_S, 1), f32),
            jax.ShapeDtypeStruct((_S, 1), f32),
            jax.ShapeDtypeStruct((1, _E), f32),
            jax.ShapeDtypeStruct((1, _E), f32),
        ],
        scratch_shapes=[pltpu.VMEM((1, _E), f32)],
    )(ctx2d, hs, Wo, g2r, Wr, noise)

    pos1, pos2, bexp = pl.pallas_call(
        _slot_kernel,
        grid=(1,),
        in_specs=[
            pl.BlockSpec((1, _E), lambda i: (0, 0)),
            pl.BlockSpec((1, _S), lambda i: (0, 0)),
            pl.BlockSpec((1, _S), lambda i: (0, 0)),
            pl.BlockSpec((1, _S), lambda i: (0, 0)),
            pl.BlockSpec((1, _S), lambda i: (0, 0)),
        ],
        out_specs=[
            pl.BlockSpec((1, _S), lambda i: (0, 0)),
            pl.BlockSpec((1, _S), lambda i: (0, 0)),
            pl.BlockSpec((1, _NBLK), lambda i: (0, 0)),
        ],
        out_shape=[
            jax.ShapeDtypeStruct((1, _S), jnp.int32),
            jax.ShapeDtypeStruct((1, _S), jnp.int32),
            jax.ShapeDtypeStruct((1, _NBLK), jnp.int32),
        ],
    )(cnt, r1, r2, e1, e2)

    xq = [x_f[:, q * _QD:(q + 1) * _QD] for q in range(4)]
    xgq = _sc_scatter(xq, pos1, pos2)

    wg_b = Wg.astype(jnp.bfloat16)
    wu_b = Wu.astype(jnp.bfloat16)
    wd_b = Wd.astype(jnp.bfloat16)

    qspec = pl.BlockSpec((_TB, _QD), lambda b, s: (b, 0))
    ygq = pl.pallas_call(
        _ffn_kernel,
        grid_spec=pltpu.PrefetchScalarGridSpec(
            num_scalar_prefetch=1,
            grid=(_NBLK,),
            in_specs=[
                qspec, qspec, qspec, qspec,
                pl.BlockSpec((1, _F, _D), lambda b, s: (s[b], 0, 0)),
                pl.BlockSpec((1, _F, _D), lambda b, s: (s[b], 0, 0)),
                pl.BlockSpec((1, _D, _F), lambda b, s: (s[b], 0, 0)),
            ],
            out_specs=[qspec, qspec, qspec, qspec],
        ),
        out_shape=[jax.ShapeDtypeStruct((_SLOTS, _QD), jnp.float32)] * 4,
    )(bexp.reshape(_NBLK), *xgq, wg_b, wu_b, wd_b)

    gq = _sc_gather(ygq, pos1, pos2)

    gspec = pl.BlockSpec((_TB, _QD), lambda i: (i, 0))
    out2d = pl.pallas_call(
        _combine_kernel,
        grid=(_NTB,),
        in_specs=[
            pl.BlockSpec((_TB, _D), lambda i: (i, 0)),
            pl.BlockSpec((_TB, 1), lambda i: (i, 0)),
            pl.BlockSpec((_TB, 1), lambda i: (i, 0)),
        ] + [gspec] * 8,
        out_specs=pl.BlockSpec((_TB, _D), lambda i: (i, 0)),
        out_shape=jax.ShapeDtypeStruct((_S, _D), f32),
    )(h_res, w1c, w2c, *gq)

    aux_loss = jnp.mean(_E * (aux_part[0] / _S) ** 2)
    return out2d.reshape(_B, _S, _D), aux_loss


# speedup vs baseline: 1.2351x; 1.0693x over previous
import jax
import jax.numpy as jnp
from jax.experimental import pallas as pl
from jax.experimental.pallas import tpu as pltpu
from jax.experimental.pallas import tpu_sc as plsc

_B, _S, _D = 1, 2048, 1024
_H, _Dh = 16, 64
_E, _K, _F = 8, 2, 1408
_EPS = 1e-06
_JITTER = 0.01
_TB = 256
_NTB = _S // _TB
_CK = 512
_NBLK = 24
_SLOTS = _NBLK * _TB
_W = 128
_QD = _D // 4


def _rms(x, g):
    v = jnp.mean(x * x, axis=-1, keepdims=True)
    return x * jax.lax.rsqrt(v + _EPS) * g


def _mm(a, b_t, out_dtype=jnp.float32):
    return jax.lax.dot_general(a, b_t, (((1,), (1,)), ((), ())),
                               preferred_element_type=out_dtype)


def _qkv_kernel(hs_ref, g1_ref, wq_ref, wk_ref, wv_ref, bq_ref, bk_ref,
                bv_ref, cos_ref, sin_ref, q_ref, k_ref, v_ref):
    x = hs_ref[...]
    h = _rms(x, g1_ref[...])
    c = cos_ref[...][:, None, :]
    s = sin_ref[...][:, None, :]
    sgn = jnp.where(
        jax.lax.broadcasted_iota(jnp.int32, (1, 1, _Dh), 2) < (_Dh // 2),
        -1.0, 1.0).astype(jnp.float32)

    def rope(y):
        y3 = y.reshape(_TB, _H, _Dh)
        rot = jnp.roll(y3, _Dh // 2, axis=-1) * sgn
        return (y3 * c + rot * s).transpose(1, 0, 2)

    q = _mm(h, wq_ref[...]) + bq_ref[...]
    k = _mm(h, wk_ref[...]) + bk_ref[...]
    v = _mm(h, wv_ref[...]) + bv_ref[...]
    q_ref[...] = rope(q)
    k_ref[...] = rope(k)
    v_ref[...] = v.reshape(_TB, _H, _Dh).transpose(1, 0, 2)


_TBQ = 512


def _attn_kernel(q_ref, k_ref, v_ref, o_ref):
    qb = pl.program_id(1)
    q = q_ref[0]
    scale = 1.0 / (_Dh ** 0.5)

    def update(s, vc, carry):
        m_p, l_p, acc_p = carry
        m_n = jnp.maximum(m_p, jnp.max(s, axis=-1, keepdims=True))
        p = jnp.exp(s - m_n)
        corr = jnp.exp(m_p - m_n)
        l_n = l_p * corr + jnp.sum(p, axis=-1, keepdims=True)
        acc_n = acc_p * corr + jax.lax.dot_general(
            p, vc, (((1,), (0,)), ((), ())), preferred_element_type=jnp.float32)
        return m_n, l_n, acc_n

    def body(c, carry):
        k = k_ref[0, pl.ds(c * _CK, _CK), :]
        v = v_ref[0, pl.ds(c * _CK, _CK), :]
        return update(_mm(q, k) * scale, v, carry)

    init = (jnp.full((_TBQ, 1), -1e30, jnp.float32),
            jnp.zeros((_TBQ, 1), jnp.float32),
            jnp.zeros((_TBQ, _Dh), jnp.float32))
    carry = jax.lax.fori_loop(0, qb, body, init)
    kd = k_ref[0, pl.ds(qb * _CK, _CK), :]
    vd = v_ref[0, pl.ds(qb * _CK, _CK), :]
    s = _mm(q, kd) * scale
    diag = (jax.lax.broadcasted_iota(jnp.int32, (_TBQ, _CK), 1)
            <= jax.lax.broadcasted_iota(jnp.int32, (_TBQ, _CK), 0))
    s = jnp.where(diag, s, jnp.float32(-1e9))
    m, l, acc = update(s, vd, carry)
    o_ref[0] = acc / l


def _post_kernel(ctx_ref, res_ref, wo_ref, g2_ref, wr_ref, noise_ref,
                 h_ref, x_ref, r1_ref, r2_ref, e1_ref, e2_ref,
                 w1_ref, w2_ref, cnt_ref, aux_ref, carry_ref):
    i = pl.program_id(0)
    attn_out = _mm(ctx_ref[...], wo_ref[...])
    h = attn_out + res_ref[...]
    h_ref[...] = h
    x = _rms(h, g2_ref[...])
    x_ref[...] = x
    logits = _mm(x, wr_ref[...]) + noise_ref[...]
    m = jnp.max(logits, axis=-1, keepdims=True)
    p = jnp.exp(logits - m)
    probs = p / jnp.sum(p, axis=-1, keepdims=True)
    lane = jax.lax.broadcasted_iota(jnp.int32, (_TB, _E), 1)
    i1 = jnp.argmax(probs, axis=-1)[:, None]
    oh1 = (lane == i1)
    v1 = jnp.max(probs, axis=-1, keepdims=True)
    probs2 = jnp.where(oh1, -1.0, probs)
    i2 = jnp.argmax(probs2, axis=-1)[:, None]
    oh2 = (lane == i2)
    v2 = jnp.max(probs2, axis=-1, keepdims=True)
    wsum = v1 + v2
    w1_ref[...] = v1 / wsum
    w2_ref[...] = v2 / wsum

    @pl.when(i == 0)
    def _():
        carry_ref[...] = jnp.zeros((1, _E), jnp.float32)

    ohb = (oh1 | oh2).astype(jnp.float32)
    ri = jax.lax.broadcasted_iota(jnp.int32, (_TB, _TB), 0)
    ci = jax.lax.broadcasted_iota(jnp.int32, (_TB, _TB), 1)
    tri = jnp.where(ri >= ci, 1.0, 0.0).astype(jnp.float32)
    incl = jax.lax.dot_general(tri, ohb, (((1,), (0,)), ((), ())),
                               preferred_element_type=jnp.float32)
    excl = incl - ohb + carry_ref[...]
    rank1 = jnp.sum(excl * oh1.astype(jnp.float32), axis=-1, keepdims=True)
    rank2 = jnp.sum(excl * oh2.astype(jnp.float32), axis=-1, keepdims=True)
    carry_ref[...] += jnp.sum(ohb, axis=0, keepdims=True)
    cnt_ref[...] = carry_ref[...]

    r1_ref[...] = rank1.reshape(1, _TB)
    r2_ref[...] = rank2.reshape(1, _TB)
    e1_ref[...] = i1.astype(jnp.float32).reshape(1, _TB)
    e2_ref[...] = i2.astype(jnp.float32).reshape(1, _TB)

    part = jnp.sum(probs, axis=0, keepdims=True)

    @pl.when(i == 0)
    def _():
        aux_ref[...] = part

    @pl.when(i > 0)
    def _():
        aux_ref[...] += part


def _slot_kernel(cnt_ref, r1_ref, r2_ref, e1_ref, e2_ref,
                 p1_ref, p2_ref, bexp_ref):
    off = []
    acc = jnp.float32(0.0)
    starts = []
    for e in range(_E):
        starts.append(acc)
        off.append(acc * _TB)
        acc = acc + jnp.ceil(cnt_ref[0, e] / _TB)
    r1 = r1_ref[...]
    r2 = r2_ref[...]
    e1 = e1_ref[...]
    e2 = e2_ref[...]
    p1 = r1
    p2 = r2
    for e in range(_E):
        p1 = p1 + jnp.where(e1 == e, off[e], 0.0)
        p2 = p2 + jnp.where(e2 == e, off[e], 0.0)
    p1_ref[...] = p1.astype(jnp.int32)
    p2_ref[...] = p2.astype(jnp.int32)
    bi = jax.lax.broadcasted_iota(jnp.int32, (1, _NBLK), 1).astype(jnp.float32)
    be = jnp.zeros((1, _NBLK), jnp.float32)
    for e in range(1, _E):
        be = be + jnp.where(bi >= starts[e], 1.0, 0.0)
    bexp_ref[...] = be.astype(jnp.int32)


def _sc_scatter(xq, pos1, pos2):
    mesh = plsc.VectorSubcoreMesh(core_axis_name="core",
                                  subcore_axis_name="subcore")
    otype = [jax.ShapeDtypeStruct((_SLOTS, _QD), jnp.float32)] * 4

    @pl.kernel(out_type=otype, mesh=mesh)
    def k(x0, x1, x2, x3, p1_hbm, p2_hbm, g0, g1, g2, g3):
        for x_hbm, xg_hbm in zip((x0, x1, x2, x3), (g0, g1, g2, g3)):
            def body(x_vmem, p1_vmem, p2_vmem, xg=xg_hbm):
                pltpu.sync_copy(x_vmem, xg.at[p1_vmem.at[0]])
                pltpu.sync_copy(x_vmem, xg.at[p2_vmem.at[0]])

            pltpu.emit_pipeline(
                body,
                grid=(_S // _W,),
                in_specs=[
                    pl.BlockSpec((_W, _QD), lambda i: (i, 0)),
                    pl.BlockSpec((1, _W), lambda i: (0, i)),
                    pl.BlockSpec((1, _W), lambda i: (0, i)),
                ],
                out_specs=[],
                core_axis_name='subcore',
                dimension_semantics=(pltpu.PARALLEL,),
            )(x_hbm, p1_hbm, p2_hbm)

    return k(*xq, pos1, pos2)


def _sc_gather(ygq, pos1, pos2):
    mesh = plsc.VectorSubcoreMesh(core_axis_name="core",
                                  subcore_axis_name="subcore")
    otype = [jax.ShapeDtypeStruct((_S, _QD), jnp.float32)] * 8

    @pl.kernel(out_type=otype, mesh=mesh)
    def k(y0, y1, y2, y3, p1_hbm, p2_hbm, *outs):
        for q, yg_hbm in enumerate((y0, y1, y2, y3)):
            for j, p_hbm in enumerate((p1_hbm, p2_hbm)):
                def body(p_vmem, g_vmem, yg=yg_hbm):
                    pltpu.sync_copy(yg.at[p_vmem.at[0]], g_vmem)

                pltpu.emit_pipeline(
                    body,
                    grid=(_S // _W,),
                    in_specs=[pl.BlockSpec((1, _W), lambda i: (0, i))],
                    out_specs=[pl.BlockSpec((_W, _QD), lambda i: (i, 0))],
                    core_axis_name='subcore',
                    dimension_semantics=(pltpu.PARALLEL,),
                )(p_hbm, outs[2 * q + j])

    return k(*ygq, pos1, pos2)


def _ffn_kernel(bexp_ref, x0_ref, x1_ref, x2_ref, x3_ref,
                wg_ref, wu_ref, wd_ref,
                y0_ref, y1_ref, y2_ref, y3_ref,
                wgb_ref, wub_ref, wdb_ref):
    b = pl.program_id(0)
    first = b == 0
    changed = bexp_ref[b] != bexp_ref[jnp.maximum(b - 1, 0)]

    @pl.when(first | changed)
    def _():
        wgb_ref[...] = wg_ref[0].astype(jnp.bfloat16)
        wub_ref[...] = wu_ref[0].astype(jnp.bfloat16)
        wdb_ref[...] = wd_ref[0].astype(jnp.bfloat16)

    x = jnp.concatenate(
        [x0_ref[...], x1_ref[...], x2_ref[...], x3_ref[...]],
        axis=1).astype(jnp.bfloat16)
    g = _mm(x, wgb_ref[...])
    u = _mm(x, wub_ref[...])
    a = (g * jax.lax.logistic(g) * u).astype(jnp.bfloat16)
    y = _mm(a, wdb_ref[...])
    y0_ref[...] = y[:, 0 * _QD:1 * _QD]
    y1_ref[...] = y[:, 1 * _QD:2 * _QD]
    y2_ref[...] = y[:, 2 * _QD:3 * _QD]
    y3_ref[...] = y[:, 3 * _QD:4 * _QD]


def _combine_kernel(h_ref, w1_ref, w2_ref, *g_refs):
    gq, out_ref = g_refs[:8], g_refs[8]
    w1 = w1_ref[...]
    w2 = w2_ref[...]
    parts = []
    for q in range(4):
        g1 = gq[2 * q][...]
        g2 = gq[2 * q + 1][...]
        parts.append(w1 * g1 + w2 * g2)
    out_ref[...] = h_ref[...] + jnp.concatenate(parts, axis=1)


def kernel(hidden_states, cos, sin, g1, g2, Wq, bq, Wk, bk, Wv, bv, Wo,
           Wr, Wg, Wu, Wd):
    hs = hidden_states.reshape(_S, _D)
    cos2 = cos.reshape(_S, _Dh)
    sin2 = sin.reshape(_S, _Dh)
    g1r = g1.reshape(1, _D)
    g2r = g2.reshape(1, _D)
    bqr = bq.reshape(1, _D)
    bkr = bk.reshape(1, _D)
    bvr = bv.reshape(1, _D)
    noise = (jax.random.normal(jax.random.key(42), (_S, _E), jnp.float32)
             * _JITTER)

    f32 = jnp.float32
    qkv_shapes = [jax.ShapeDtypeStruct((_H, _S, _Dh), f32)] * 3
    q, k, v = pl.pallas_call(
        _qkv_kernel,
        grid=(_NTB,),
        in_specs=[
            pl.BlockSpec((_TB, _D), lambda i: (i, 0)),
            pl.BlockSpec((1, _D), lambda i: (0, 0)),
            pl.BlockSpec((_D, _D), lambda i: (0, 0)),
            pl.BlockSpec((_D, _D), lambda i: (0, 0)),
            pl.BlockSpec((_D, _D), lambda i: (0, 0)),
            pl.BlockSpec((1, _D), lambda i: (0, 0)),
            pl.BlockSpec((1, _D), lambda i: (0, 0)),
            pl.BlockSpec((1, _D), lambda i: (0, 0)),
            pl.BlockSpec((_TB, _Dh), lambda i: (i, 0)),
            pl.BlockSpec((_TB, _Dh), lambda i: (i, 0)),
        ],
        out_specs=[pl.BlockSpec((_H, _TB, _Dh), lambda i: (0, i, 0))] * 3,
        out_shape=qkv_shapes,
    )(hs, g1r, Wq, Wk, Wv, bqr, bkr, bvr, cos2, sin2)

    ctx = pl.pallas_call(
        _attn_kernel,
        grid=(_H, _S // _TBQ),
        in_specs=[
            pl.BlockSpec((1, _TBQ, _Dh), lambda h, i: (h, i, 0)),
            pl.BlockSpec((1, _S, _Dh), lambda h, i: (h, 0, 0)),
            pl.BlockSpec((1, _S, _Dh), lambda h, i: (h, 0, 0)),
        ],
        out_specs=pl.BlockSpec((1, _TBQ, _Dh), lambda h, i: (h, i, 0)),
        out_shape=jax.ShapeDtypeStruct((_H, _S, _Dh), f32),
    )(q, k, v)

    ctx2d = ctx.transpose(1, 0, 2).reshape(_S, _D)

    (h_res, x_f, r1, r2, e1, e2, w1c, w2c, cnt, aux_part) = pl.pallas_call(
        _post_kernel,
        grid=(_NTB,),
        in_specs=[
            pl.BlockSpec((_TB, _D), lambda i: (i, 0)),
            pl.BlockSpec((_TB, _D), lambda i: (i, 0)),
            pl.BlockSpec((_D, _D), lambda i: (0, 0)),
            pl.BlockSpec((1, _D), lambda i: (0, 0)),
            pl.BlockSpec((_E, _D), lambda i: (0, 0)),
            pl.BlockSpec((_TB, _E), lambda i: (i, 0)),
        ],
        out_specs=[
            pl.BlockSpec((_TB, _D), lambda i: (i, 0)),
            pl.BlockSpec((_TB, _D), lambda i: (i, 0)),
            pl.BlockSpec((1, _TB), lambda i: (0, i)),
            pl.BlockSpec((1, _TB), lambda i: (0, i)),
            pl.BlockSpec((1, _TB), lambda i: (0, i)),
            pl.BlockSpec((1, _TB), lambda i: (0, i)),
            pl.BlockSpec((_TB, 1), lambda i: (i, 0)),
            pl.BlockSpec((_TB, 1), lambda i: (i, 0)),
            pl.BlockSpec((1, _E), lambda i: (0, 0)),
            pl.BlockSpec((1, _E), lambda i: (0, 0)),
        ],
        out_shape=[
            jax.ShapeDtypeStruct((_S, _D), f32),
            jax.ShapeDtypeStruct((_S, _D), f32),
            jax.ShapeDtypeStruct((1, _S), f32),
            jax.ShapeDtypeStruct((1, _S), f32),
            jax.ShapeDtypeStruct((1, _S), f32),
            jax.ShapeDtypeStruct((1, _S), f32),
            jax.ShapeDtypeStruct((_S, 1), f32),
            jax.ShapeDtypeStruct((_S, 1), f32),
            jax.ShapeDtypeStruct((1, _E), f32),
            jax.ShapeDtypeStruct((1, _E), f32),
        ],
        scratch_shapes=[pltpu.VMEM((1, _E), f32)],
    )(ctx2d, hs, Wo, g2r, Wr, noise)

    pos1, pos2, bexp = pl.pallas_call(
        _slot_kernel,
        grid=(1,),
        in_specs=[
            pl.BlockSpec((1, _E), lambda i: (0, 0)),
            pl.BlockSpec((1, _S), lambda i: (0, 0)),
            pl.BlockSpec((1, _S), lambda i: (0, 0)),
            pl.BlockSpec((1, _S), lambda i: (0, 0)),
            pl.BlockSpec((1, _S), lambda i: (0, 0)),
        ],
        out_specs=[
            pl.BlockSpec((1, _S), lambda i: (0, 0)),
            pl.BlockSpec((1, _S), lambda i: (0, 0)),
            pl.BlockSpec((1, _NBLK), lambda i: (0, 0)),
        ],
        out_shape=[
            jax.ShapeDtypeStruct((1, _S), jnp.int32),
            jax.ShapeDtypeStruct((1, _S), jnp.int32),
            jax.ShapeDtypeStruct((1, _NBLK), jnp.int32),
        ],
    )(cnt, r1, r2, e1, e2)

    xq = [x_f[:, q * _QD:(q + 1) * _QD] for q in range(4)]
    xgq = _sc_scatter(xq, pos1, pos2)

    qspec = pl.BlockSpec((_TB, _QD), lambda b, s: (b, 0))
    ygq = pl.pallas_call(
        _ffn_kernel,
        grid_spec=pltpu.PrefetchScalarGridSpec(
            num_scalar_prefetch=1,
            grid=(_NBLK,),
            in_specs=[
                qspec, qspec, qspec, qspec,
                pl.BlockSpec((1, _F, _D), lambda b, s: (s[b], 0, 0)),
                pl.BlockSpec((1, _F, _D), lambda b, s: (s[b], 0, 0)),
                pl.BlockSpec((1, _D, _F), lambda b, s: (s[b], 0, 0)),
            ],
            out_specs=[qspec, qspec, qspec, qspec],
            scratch_shapes=[pltpu.VMEM((_F, _D), jnp.bfloat16),
                            pltpu.VMEM((_F, _D), jnp.bfloat16),
                            pltpu.VMEM((_D, _F), jnp.bfloat16)],
        ),
        out_shape=[jax.ShapeDtypeStruct((_SLOTS, _QD), jnp.float32)] * 4,
    )(bexp.reshape(_NBLK), *xgq, Wg, Wu, Wd)

    gq = _sc_gather(ygq, pos1, pos2)

    gspec = pl.BlockSpec((_TB, _QD), lambda i: (i, 0))
    out2d = pl.pallas_call(
        _combine_kernel,
        grid=(_NTB,),
        in_specs=[
            pl.BlockSpec((_TB, _D), lambda i: (i, 0)),
            pl.BlockSpec((_TB, 1), lambda i: (i, 0)),
            pl.BlockSpec((_TB, 1), lambda i: (i, 0)),
        ] + [gspec] * 8,
        out_specs=pl.BlockSpec((_TB, _D), lambda i: (i, 0)),
        out_shape=jax.ShapeDtypeStruct((_S, _D), f32),
    )(h_res, w1c, w2c, *gq)

    aux_loss = jnp.mean(_E * (aux_part[0] / _S) ** 2)
    return out2d.reshape(_B, _S, _D), aux_loss


# speedup vs baseline: 1.2558x; 1.0168x over previous
import jax
import jax.numpy as jnp
from jax.experimental import pallas as pl
from jax.experimental.pallas import tpu as pltpu
from jax.experimental.pallas import tpu_sc as plsc

_B, _S, _D = 1, 2048, 1024
_H, _Dh = 16, 64
_E, _K, _F = 8, 2, 1408
_EPS = 1e-06
_JITTER = 0.01
_TB = 256
_NTB = _S // _TB
_CK = 512
_NBLK = 24
_SLOTS = _NBLK * _TB
_W = 128
_QD = _D // 4


def _rms(x, g):
    v = jnp.mean(x * x, axis=-1, keepdims=True)
    return x * jax.lax.rsqrt(v + _EPS) * g


def _mm(a, b_t, out_dtype=jnp.float32):
    return jax.lax.dot_general(a, b_t, (((1,), (1,)), ((), ())),
                               preferred_element_type=out_dtype)


def _qkv_kernel(hs_ref, g1_ref, wq_ref, wk_ref, wv_ref, bq_ref, bk_ref,
                bv_ref, cos_ref, sin_ref, q_ref, k_ref, v_ref):
    x = hs_ref[...]
    h = _rms(x, g1_ref[...])
    c = cos_ref[...][:, None, :]
    s = sin_ref[...][:, None, :]
    sgn = jnp.where(
        jax.lax.broadcasted_iota(jnp.int32, (1, 1, _Dh), 2) < (_Dh // 2),
        -1.0, 1.0).astype(jnp.float32)

    def rope(y):
        y3 = y.reshape(_TB, _H, _Dh)
        rot = jnp.roll(y3, _Dh // 2, axis=-1) * sgn
        return (y3 * c + rot * s).transpose(1, 0, 2)

    q = _mm(h, wq_ref[...]) + bq_ref[...]
    k = _mm(h, wk_ref[...]) + bk_ref[...]
    v = _mm(h, wv_ref[...]) + bv_ref[...]
    q_ref[...] = rope(q)
    k_ref[...] = rope(k)
    v_ref[...] = v.reshape(_TB, _H, _Dh).transpose(1, 0, 2)


_TBQ = 512


def _attn_kernel(q_ref, k_ref, v_ref, o_ref):
    qb = pl.program_id(1)
    q = q_ref[0]
    scale = 1.0 / (_Dh ** 0.5)

    def update(s, vc, carry):
        m_p, l_p, acc_p = carry
        m_n = jnp.maximum(m_p, jnp.max(s, axis=-1, keepdims=True))
        p = jnp.exp(s - m_n)
        corr = jnp.exp(m_p - m_n)
        l_n = l_p * corr + jnp.sum(p, axis=-1, keepdims=True)
        acc_n = acc_p * corr + jax.lax.dot_general(
            p, vc, (((1,), (0,)), ((), ())), preferred_element_type=jnp.float32)
        return m_n, l_n, acc_n

    def body(c, carry):
        k = k_ref[0, pl.ds(c * _CK, _CK), :]
        v = v_ref[0, pl.ds(c * _CK, _CK), :]
        return update(_mm(q, k) * scale, v, carry)

    init = (jnp.full((_TBQ, 1), -1e30, jnp.float32),
            jnp.zeros((_TBQ, 1), jnp.float32),
            jnp.zeros((_TBQ, _Dh), jnp.float32))
    carry = jax.lax.fori_loop(0, qb, body, init)
    kd = k_ref[0, pl.ds(qb * _CK, _CK), :]
    vd = v_ref[0, pl.ds(qb * _CK, _CK), :]
    s = _mm(q, kd) * scale
    diag = (jax.lax.broadcasted_iota(jnp.int32, (_TBQ, _CK), 1)
            <= jax.lax.broadcasted_iota(jnp.int32, (_TBQ, _CK), 0))
    s = jnp.where(diag, s, jnp.float32(-1e9))
    m, l, acc = update(s, vd, carry)
    o_ref[0] = acc / l


def _post_kernel(ctx_ref, res_ref, wo_ref, g2_ref, wr_ref, noise_ref,
                 h_ref, x_ref, r1_ref, r2_ref, e1_ref, e2_ref,
                 w1_ref, w2_ref, cnt_ref, aux_ref, carry_ref):
    i = pl.program_id(0)
    ctx2d = ctx_ref[...].transpose(1, 0, 2).reshape(_TB, _D)
    attn_out = _mm(ctx2d, wo_ref[...])
    h = attn_out + res_ref[...]
    h_ref[...] = h
    x = _rms(h, g2_ref[...])
    x_ref[...] = x
    logits = _mm(x, wr_ref[...]) + noise_ref[...]
    m = jnp.max(logits, axis=-1, keepdims=True)
    p = jnp.exp(logits - m)
    probs = p / jnp.sum(p, axis=-1, keepdims=True)
    lane = jax.lax.broadcasted_iota(jnp.int32, (_TB, _E), 1)
    i1 = jnp.argmax(probs, axis=-1)[:, None]
    oh1 = (lane == i1)
    v1 = jnp.max(probs, axis=-1, keepdims=True)
    probs2 = jnp.where(oh1, -1.0, probs)
    i2 = jnp.argmax(probs2, axis=-1)[:, None]
    oh2 = (lane == i2)
    v2 = jnp.max(probs2, axis=-1, keepdims=True)
    wsum = v1 + v2
    w1_ref[...] = v1 / wsum
    w2_ref[...] = v2 / wsum

    @pl.when(i == 0)
    def _():
        carry_ref[...] = jnp.zeros((1, _E), jnp.float32)

    ohb = (oh1 | oh2).astype(jnp.float32)
    ri = jax.lax.broadcasted_iota(jnp.int32, (_TB, _TB), 0)
    ci = jax.lax.broadcasted_iota(jnp.int32, (_TB, _TB), 1)
    tri = jnp.where(ri >= ci, 1.0, 0.0).astype(jnp.float32)
    incl = jax.lax.dot_general(tri, ohb, (((1,), (0,)), ((), ())),
                               preferred_element_type=jnp.float32)
    excl = incl - ohb + carry_ref[...]
    rank1 = jnp.sum(excl * oh1.astype(jnp.float32), axis=-1, keepdims=True)
    rank2 = jnp.sum(excl * oh2.astype(jnp.float32), axis=-1, keepdims=True)
    carry_ref[...] += jnp.sum(ohb, axis=0, keepdims=True)
    cnt_ref[...] = carry_ref[...]

    r1_ref[...] = rank1.reshape(1, _TB)
    r2_ref[...] = rank2.reshape(1, _TB)
    e1_ref[...] = i1.astype(jnp.float32).reshape(1, _TB)
    e2_ref[...] = i2.astype(jnp.float32).reshape(1, _TB)

    part = jnp.sum(probs, axis=0, keepdims=True)

    @pl.when(i == 0)
    def _():
        aux_ref[...] = part

    @pl.when(i > 0)
    def _():
        aux_ref[...] += part


def _slot_kernel(cnt_ref, r1_ref, r2_ref, e1_ref, e2_ref,
                 p1_ref, p2_ref, bexp_ref):
    off = []
    acc = jnp.float32(0.0)
    starts = []
    for e in range(_E):
        starts.append(acc)
        off.append(acc * _TB)
        acc = acc + jnp.ceil(cnt_ref[0, e] / _TB)
    r1 = r1_ref[...]
    r2 = r2_ref[...]
    e1 = e1_ref[...]
    e2 = e2_ref[...]
    p1 = r1
    p2 = r2
    for e in range(_E):
        p1 = p1 + jnp.where(e1 == e, off[e], 0.0)
        p2 = p2 + jnp.where(e2 == e, off[e], 0.0)
    p1_ref[...] = p1.astype(jnp.int32)
    p2_ref[...] = p2.astype(jnp.int32)
    bi = jax.lax.broadcasted_iota(jnp.int32, (1, _NBLK), 1).astype(jnp.float32)
    be = jnp.zeros((1, _NBLK), jnp.float32)
    for e in range(1, _E):
        be = be + jnp.where(bi >= starts[e], 1.0, 0.0)
    bexp_ref[...] = be.astype(jnp.int32)


def _sc_scatter(xq, pos1, pos2):
    mesh = plsc.VectorSubcoreMesh(core_axis_name="core",
                                  subcore_axis_name="subcore")
    otype = [jax.ShapeDtypeStruct((_SLOTS, _QD), jnp.float32)] * 4

    @pl.kernel(out_type=otype, mesh=mesh)
    def k(x0, x1, x2, x3, p1_hbm, p2_hbm, g0, g1, g2, g3):
        for x_hbm, xg_hbm in zip((x0, x1, x2, x3), (g0, g1, g2, g3)):
            def body(x_vmem, p1_vmem, p2_vmem, xg=xg_hbm):
                pltpu.sync_copy(x_vmem, xg.at[p1_vmem.at[0]])
                pltpu.sync_copy(x_vmem, xg.at[p2_vmem.at[0]])

            pltpu.emit_pipeline(
                body,
                grid=(_S // _W,),
                in_specs=[
                    pl.BlockSpec((_W, _QD), lambda i: (i, 0)),
                    pl.BlockSpec((1, _W), lambda i: (0, i)),
                    pl.BlockSpec((1, _W), lambda i: (0, i)),
                ],
                out_specs=[],
                core_axis_name='subcore',
                dimension_semantics=(pltpu.PARALLEL,),
            )(x_hbm, p1_hbm, p2_hbm)

    return k(*xq, pos1, pos2)


def _sc_gather(ygq, pos1, pos2):
    mesh = plsc.VectorSubcoreMesh(core_axis_name="core",
                                  subcore_axis_name="subcore")
    otype = [jax.ShapeDtypeStruct((_S, _QD), jnp.float32)] * 8

    @pl.kernel(out_type=otype, mesh=mesh)
    def k(y0, y1, y2, y3, p1_hbm, p2_hbm, *outs):
        for q, yg_hbm in enumerate((y0, y1, y2, y3)):
            for j, p_hbm in enumerate((p1_hbm, p2_hbm)):
                def body(p_vmem, g_vmem, yg=yg_hbm):
                    pltpu.sync_copy(yg.at[p_vmem.at[0]], g_vmem)

                pltpu.emit_pipeline(
                    body,
                    grid=(_S // _W,),
                    in_specs=[pl.BlockSpec((1, _W), lambda i: (0, i))],
                    out_specs=[pl.BlockSpec((_W, _QD), lambda i: (i, 0))],
                    core_axis_name='subcore',
                    dimension_semantics=(pltpu.PARALLEL,),
                )(p_hbm, outs[2 * q + j])

    return k(*ygq, pos1, pos2)


def _ffn_kernel(bexp_ref, x0_ref, x1_ref, x2_ref, x3_ref,
                wg_ref, wu_ref, wd_ref,
                y0_ref, y1_ref, y2_ref, y3_ref,
                wgb_ref, wub_ref, wdb_ref):
    b = pl.program_id(0)
    first = b == 0
    changed = bexp_ref[b] != bexp_ref[jnp.maximum(b - 1, 0)]

    @pl.when(first | changed)
    def _():
        wgb_ref[...] = wg_ref[0].astype(jnp.bfloat16)
        wub_ref[...] = wu_ref[0].astype(jnp.bfloat16)
        wdb_ref[...] = wd_ref[0].astype(jnp.bfloat16)

    x = jnp.concatenate(
        [x0_ref[...], x1_ref[...], x2_ref[...], x3_ref[...]],
        axis=1).astype(jnp.bfloat16)
    g = _mm(x, wgb_ref[...])
    u = _mm(x, wub_ref[...])
    a = (g * jax.lax.logistic(g) * u).astype(jnp.bfloat16)
    y = _mm(a, wdb_ref[...])
    y0_ref[...] = y[:, 0 * _QD:1 * _QD]
    y1_ref[...] = y[:, 1 * _QD:2 * _QD]
    y2_ref[...] = y[:, 2 * _QD:3 * _QD]
    y3_ref[...] = y[:, 3 * _QD:4 * _QD]


def _combine_kernel(h_ref, w1_ref, w2_ref, *g_refs):
    gq, out_ref = g_refs[:8], g_refs[8]
    w1 = w1_ref[...]
    w2 = w2_ref[...]
    parts = []
    for q in range(4):
        g1 = gq[2 * q][...]
        g2 = gq[2 * q + 1][...]
        parts.append(w1 * g1 + w2 * g2)
    out_ref[...] = h_ref[...] + jnp.concatenate(parts, axis=1)


def kernel(hidden_states, cos, sin, g1, g2, Wq, bq, Wk, bk, Wv, bv, Wo,
           Wr, Wg, Wu, Wd):
    hs = hidden_states.reshape(_S, _D)
    cos2 = cos.reshape(_S, _Dh)
    sin2 = sin.reshape(_S, _Dh)
    g1r = g1.reshape(1, _D)
    g2r = g2.reshape(1, _D)
    bqr = bq.reshape(1, _D)
    bkr = bk.reshape(1, _D)
    bvr = bv.reshape(1, _D)
    noise = (jax.random.normal(jax.random.key(42), (_S, _E), jnp.float32)
             * _JITTER)

    f32 = jnp.float32
    qkv_shapes = [jax.ShapeDtypeStruct((_H, _S, _Dh), f32)] * 3
    q, k, v = pl.pallas_call(
        _qkv_kernel,
        grid=(_NTB,),
        in_specs=[
            pl.BlockSpec((_TB, _D), lambda i: (i, 0)),
            pl.BlockSpec((1, _D), lambda i: (0, 0)),
            pl.BlockSpec((_D, _D), lambda i: (0, 0)),
            pl.BlockSpec((_D, _D), lambda i: (0, 0)),
            pl.BlockSpec((_D, _D), lambda i: (0, 0)),
            pl.BlockSpec((1, _D), lambda i: (0, 0)),
            pl.BlockSpec((1, _D), lambda i: (0, 0)),
            pl.BlockSpec((1, _D), lambda i: (0, 0)),
            pl.BlockSpec((_TB, _Dh), lambda i: (i, 0)),
            pl.BlockSpec((_TB, _Dh), lambda i: (i, 0)),
        ],
        out_specs=[pl.BlockSpec((_H, _TB, _Dh), lambda i: (0, i, 0))] * 3,
        out_shape=qkv_shapes,
    )(hs, g1r, Wq, Wk, Wv, bqr, bkr, bvr, cos2, sin2)

    ctx = pl.pallas_call(
        _attn_kernel,
        grid=(_H, _S // _TBQ),
        in_specs=[
            pl.BlockSpec((1, _TBQ, _Dh), lambda h, i: (h, i, 0)),
            pl.BlockSpec((1, _S, _Dh), lambda h, i: (h, 0, 0)),
            pl.BlockSpec((1, _S, _Dh), lambda h, i: (h, 0, 0)),
        ],
        out_specs=pl.BlockSpec((1, _TBQ, _Dh), lambda h, i: (h, i, 0)),
        out_shape=jax.ShapeDtypeStruct((_H, _S, _Dh), f32),
    )(q, k, v)

    (h_res, x_f, r1, r2, e1, e2, w1c, w2c, cnt, aux_part) = pl.pallas_call(
        _post_kernel,
        grid=(_NTB,),
        in_specs=[
            pl.BlockSpec((_H, _TB, _Dh), lambda i: (0, i, 0)),
            pl.BlockSpec((_TB, _D), lambda i: (i, 0)),
            pl.BlockSpec((_D, _D), lambda i: (0, 0)),
            pl.BlockSpec((1, _D), lambda i: (0, 0)),
            pl.BlockSpec((_E, _D), lambda i: (0, 0)),
            pl.BlockSpec((_TB, _E), lambda i: (i, 0)),
        ],
        out_specs=[
            pl.BlockSpec((_TB, _D), lambda i: (i, 0)),
            pl.BlockSpec((_TB, _D), lambda i: (i, 0)),
            pl.BlockSpec((1, _TB), lambda i: (0, i)),
            pl.BlockSpec((1, _TB), lambda i: (0, i)),
            pl.BlockSpec((1, _TB), lambda i: (0, i)),
            pl.BlockSpec((1, _TB), lambda i: (0, i)),
            pl.BlockSpec((_TB, 1), lambda i: (i, 0)),
            pl.BlockSpec((_TB, 1), lambda i: (i, 0)),
            pl.BlockSpec((1, _E), lambda i: (0, 0)),
            pl.BlockSpec((1, _E), lambda i: (0, 0)),
        ],
        out_shape=[
            jax.ShapeDtypeStruct((_S, _D), f32),
            jax.ShapeDtypeStruct((_S, _D), f32),
            jax.ShapeDtypeStruct((1, _S), f32),
            jax.ShapeDtypeStruct((1, _S), f32),
            jax.ShapeDtypeStruct((1, _S), f32),
            jax.ShapeDtypeStruct((1, _S), f32),
            jax.ShapeDtypeStruct((_S, 1), f32),
            jax.ShapeDtypeStruct((_S, 1), f32),
            jax.ShapeDtypeStruct((1, _E), f32),
            jax.ShapeDtypeStruct((1, _E), f32),
        ],
        scratch_shapes=[pltpu.VMEM((1, _E), f32)],
    )(ctx, hs, Wo, g2r, Wr, noise)

    pos1, pos2, bexp = pl.pallas_call(
        _slot_kernel,
        grid=(1,),
        in_specs=[
            pl.BlockSpec((1, _E), lambda i: (0, 0)),
            pl.BlockSpec((1, _S), lambda i: (0, 0)),
            pl.BlockSpec((1, _S), lambda i: (0, 0)),
            pl.BlockSpec((1, _S), lambda i: (0, 0)),
            pl.BlockSpec((1, _S), lambda i: (0, 0)),
        ],
        out_specs=[
            pl.BlockSpec((1, _S), lambda i: (0, 0)),
            pl.BlockSpec((1, _S), lambda i: (0, 0)),
            pl.BlockSpec((1, _NBLK), lambda i: (0, 0)),
        ],
        out_shape=[
            jax.ShapeDtypeStruct((1, _S), jnp.int32),
            jax.ShapeDtypeStruct((1, _S), jnp.int32),
            jax.ShapeDtypeStruct((1, _NBLK), jnp.int32),
        ],
    )(cnt, r1, r2, e1, e2)

    xq = [x_f[:, q * _QD:(q + 1) * _QD] for q in range(4)]
    xgq = _sc_scatter(xq, pos1, pos2)

    qspec = pl.BlockSpec((_TB, _QD), lambda b, s: (b, 0))
    ygq = pl.pallas_call(
        _ffn_kernel,
        grid_spec=pltpu.PrefetchScalarGridSpec(
            num_scalar_prefetch=1,
            grid=(_NBLK,),
            in_specs=[
                qspec, qspec, qspec, qspec,
                pl.BlockSpec((1, _F, _D), lambda b, s: (s[b], 0, 0)),
                pl.BlockSpec((1, _F, _D), lambda b, s: (s[b], 0, 0)),
                pl.BlockSpec((1, _D, _F), lambda b, s: (s[b], 0, 0)),
            ],
            out_specs=[qspec, qspec, qspec, qspec],
            scratch_shapes=[pltpu.VMEM((_F, _D), jnp.bfloat16),
                            pltpu.VMEM((_F, _D), jnp.bfloat16),
                            pltpu.VMEM((_D, _F), jnp.bfloat16)],
        ),
        out_shape=[jax.ShapeDtypeStruct((_SLOTS, _QD), jnp.float32)] * 4,
    )(bexp.reshape(_NBLK), *xgq, Wg, Wu, Wd)

    gq = _sc_gather(ygq, pos1, pos2)

    gspec = pl.BlockSpec((_TB, _QD), lambda i: (i, 0))
    out2d = pl.pallas_call(
        _combine_kernel,
        grid=(_NTB,),
        in_specs=[
            pl.BlockSpec((_TB, _D), lambda i: (i, 0)),
            pl.BlockSpec((_TB, 1), lambda i: (i, 0)),
            pl.BlockSpec((_TB, 1), lambda i: (i, 0)),
        ] + [gspec] * 8,
        out_specs=pl.BlockSpec((_TB, _D), lambda i: (i, 0)),
        out_shape=jax.ShapeDtypeStruct((_S, _D), f32),
    )(h_res, w1c, w2c, *gq)

    aux_loss = jnp.mean(_E * (aux_part[0] / _S) ** 2)
    return out2d.reshape(_B, _S, _D), aux_loss
